# Initial kernel scaffold; baseline (speedup 1.0000x reference)
import jax
import jax.numpy as jnp
from jax.experimental import pallas as pl


def kernel(x):
    raise NotImplementedError("write your pallas kernel here")


# speedup vs baseline: 2.4365x; 2.4365x over previous
import jax
import jax.numpy as jnp
from jax.experimental import pallas as pl
from jax.experimental.pallas import tpu as pltpu

INTERVAL_ = 4
BLOCK_ROWS = 512


def _body(x_ref, o_ref):
    rows = jax.lax.broadcasted_iota(jnp.int32, x_ref.shape, 0)
    keep = (rows % INTERVAL_) != 0
    o_ref[...] = jnp.where(keep, x_ref[...], 0.0)


def kernel(x):
    n, d = x.shape
    grid = (n // BLOCK_ROWS,)
    return pl.pallas_call(
        _body,
        grid=grid,
        in_specs=[pl.BlockSpec((BLOCK_ROWS, d), lambda i: (i, 0))],
        out_specs=pl.BlockSpec((BLOCK_ROWS, d), lambda i: (i, 0)),
        out_shape=jax.ShapeDtypeStruct((n, d), x.dtype),
    )(x)
